# baseline (speedup 1.0000x reference)
import functools

import jax
import jax.numpy as jnp
from jax import lax
from jax.experimental import pallas as pl
from jax.experimental.pallas import tpu as pltpu
from jax.experimental.pallas import tpu_sc as plsc

NVERTS = 100000
BAND = 3072
REM = NVERTS - 32 * BAND
NG = 24
L_ = 512


def kernel(x, vs):
    B, L, D = x.shape
    R = D * B
    xt = jnp.transpose(x, (2, 0, 1)).reshape(R, L)
    mesh = plsc.VectorSubcoreMesh(core_axis_name="c", subcore_axis_name="s")

    @functools.partial(
        pl.kernel,
        mesh=mesh,
        out_type=jax.ShapeDtypeStruct((R, NVERTS), jnp.float32),
        scratch_types=[
            pltpu.VMEM((8, BAND), jnp.float32),
            pltpu.VMEM((8, REM), jnp.float32),
            pltpu.VMEM((8, BAND - L_), jnp.float32),
            pltpu.VMEM((2, 8, L_), jnp.float32),
            pltpu.SemaphoreType.DMA,
            pltpu.SemaphoreType.DMA,
            pltpu.SemaphoreType.DMA,
        ],
    )
    def sc_k(xt_hbm, vs_hbm, out_hbm, zband, zrem, zht, xrows,
             dsem, hsem, xsem):
        wid = lax.axis_index("s") * 2 + lax.axis_index("c")
        col0 = pl.multiple_of(wid * BAND, 128)

        zero16 = jnp.zeros((16,), jnp.float32)

        for rr in range(8):
            zb = zband.at[rr]
            zr = zrem.at[rr]
            zh = zht.at[rr]

            def zf_band(i, c):
                zb[pl.ds(i * 16, 16)] = zero16
                return c

            lax.fori_loop(0, BAND // 16, zf_band, 0, unroll=8)

            def zf_rem(i, c):
                zr[pl.ds(i * 16, 16)] = zero16
                return c

            lax.fori_loop(0, REM // 16, zf_rem, 0, unroll=8)

            def zf_ht(i, c):
                zh[pl.ds(i * 16, 16)] = zero16
                return c

            lax.fori_loop(0, (BAND - L_) // 16, zf_ht, 0, unroll=8)

        @pl.when(wid != 0)
        def _plain_bands():
            def band_g(g, c):
                r0 = pl.multiple_of(g * 8, 8)
                pltpu.async_copy(
                    zband,
                    out_hbm.at[pl.ds(r0, 8), pl.ds(col0, BAND)], dsem)
                return c

            lax.fori_loop(0, NG, band_g, 0)

        @pl.when(wid == 0)
        def _scatter_bands():
            pltpu.async_copy(xt_hbm.at[pl.ds(0, 8)], xrows.at[0], xsem)

            def group(g, c):
                r0 = pl.multiple_of(g * 8, 8)
                pg = g % 2
                pltpu.make_async_copy(
                    xt_hbm.at[pl.ds(0, 8)], xrows.at[0], xsem).wait()

                pltpu.async_copy(
                    xrows.at[pg],
                    out_hbm.at[pl.ds(r0, 8), pl.ds(0, L_)], hsem)
                pltpu.async_copy(
                    zht,
                    out_hbm.at[pl.ds(r0, 8), pl.ds(L_, BAND - L_)], dsem)

                @pl.when(g >= 1)
                def _drain_prev_head():
                    pltpu.make_async_copy(
                        xrows.at[0],
                        out_hbm.at[pl.ds(0, 8), pl.ds(0, L_)], hsem).wait()

                @pl.when(g + 1 < NG)
                def _prefetch():
                    r1 = pl.multiple_of((g + 1) * 8, 8)
                    pltpu.async_copy(
                        xt_hbm.at[pl.ds(r1, 8)], xrows.at[1 - pg], xsem)
                return c

            lax.fori_loop(0, NG, group, 0)
            pltpu.make_async_copy(
                xrows.at[0],
                out_hbm.at[pl.ds(0, 8), pl.ds(0, L_)], hsem).wait()

        @pl.when(wid >= 8)
        def _rem_band():
            r0 = pl.multiple_of((wid - 8) * 8, 8)
            pltpu.async_copy(
                zrem,
                out_hbm.at[pl.ds(r0, 8), pl.ds(32 * BAND, REM)], dsem)

        @pl.when(wid != 0)
        def _drain_bands():
            def dr(g, c):
                pltpu.make_async_copy(
                    zband,
                    out_hbm.at[pl.ds(0, 8), pl.ds(0, BAND)], dsem).wait()
                return c

            lax.fori_loop(0, NG, dr, 0)

        @pl.when(wid == 0)
        def _drain_tails():
            def dr(g, c):
                pltpu.make_async_copy(
                    zht,
                    out_hbm.at[pl.ds(0, 8), pl.ds(L_, BAND - L_)],
                    dsem).wait()
                return c

            lax.fori_loop(0, NG, dr, 0)

        @pl.when(wid >= 8)
        def _drain_rem():
            pltpu.make_async_copy(
                zrem,
                out_hbm.at[pl.ds(0, 8), pl.ds(32 * BAND, REM)], dsem).wait()

    out = sc_k(xt, vs)
    return jnp.transpose(out.reshape(D, B, NVERTS), (1, 2, 0))
